# Initial kernel scaffold; baseline (speedup 1.0000x reference)
import jax
import jax.numpy as jnp
from jax.experimental import pallas as pl


def kernel(x, table, W, b):
    raise NotImplementedError("write your pallas kernel here")


# speedup vs baseline: 4.3220x; 4.3220x over previous
import functools

import jax
import jax.numpy as jnp
from jax import lax
from jax.experimental import pallas as pl
from jax.experimental.pallas import tpu as pltpu
from jax.experimental.pallas import tpu_sc as plsc

B, C, L = 256, 64, 512
E = 16
NPAT = 1000000
WIN = 4
PAD = WIN // 2

NC, NS = 2, 16
NW = NC * NS
ROWS = B * L
RPW = ROWS // NW
CHUNK = 128
NCHUNK = RPW // CHUNK
GROUP = 8


def _hash_indices(x):
    xp = jnp.pad(x, ((0, 0), (0, 0), (PAD, PAD)), mode="edge")
    windows = jnp.stack([xp[:, :, j:j + L] for j in range(WIN)], axis=-1)
    hashed = (windows * 31.0).sum(axis=(1, 3))
    idx = hashed.astype(jnp.int64) % NPAT
    idx = jnp.clip(idx, 0, NPAT - 1)
    return idx.astype(jnp.int32)


def _sc_gather(table, idx3):
    mesh = plsc.VectorSubcoreMesh(core_axis_name="c", subcore_axis_name="s")

    @functools.partial(
        pl.kernel,
        out_type=jax.ShapeDtypeStruct((ROWS, E), jnp.float32),
        mesh=mesh,
        scratch_types=[
            pltpu.VMEM((NCHUNK, CHUNK), jnp.int32),
            pltpu.VMEM((RPW, E), jnp.float32),
            pltpu.SemaphoreType.DMA,
        ],
        compiler_params=pltpu.CompilerParams(use_tc_tiling_on_sc=False),
    )
    def gather_k(table_hbm, idx_hbm, out_hbm, idx_v, rows_v, sem):
        wid = lax.axis_index("s") * NC + lax.axis_index("c")
        pltpu.sync_copy(idx_hbm.at[wid], idx_v)

        def group_body(g, _):
            base_c = g * GROUP
            copies = []
            for k in range(GROUP):
                c = base_c + k
                cp = pltpu.make_async_copy(
                    table_hbm.at[idx_v.at[c]],
                    rows_v.at[pl.ds(c * CHUNK, CHUNK), :],
                    sem,
                )
                cp.start()
                copies.append(cp)
            for cp in copies:
                cp.wait()
            return 0

        lax.fori_loop(0, NCHUNK // GROUP, group_body, 0, unroll=False)
        pltpu.sync_copy(rows_v, out_hbm.at[pl.ds(wid * RPW, RPW), :])

    return gather_k(table, idx3)


def _tc_project(emb3, W, b):
    def proj_body(emb_ref, w_ref, b_ref, out_ref):
        e = emb_ref[0]
        w = w_ref[...]
        r = lax.dot_general(w, e, (((1,), (1,)), ((), ())),
                            preferred_element_type=jnp.float32)
        out_ref[0] = r + b_ref[...]

    return pl.pallas_call(
        proj_body,
        grid=(B,),
        in_specs=[
            pl.BlockSpec((1, L, E), lambda i: (i, 0, 0)),
            pl.BlockSpec((C, E), lambda i: (0, 0)),
            pl.BlockSpec((C, 1), lambda i: (0, 0)),
        ],
        out_specs=pl.BlockSpec((1, C, L), lambda i: (i, 0, 0)),
        out_shape=jax.ShapeDtypeStruct((B, C, L), jnp.float32),
    )(emb3, W, b.reshape(C, 1))


def kernel(x, table, W, b):
    idx = _hash_indices(x)
    idx3 = idx.reshape(NW, NCHUNK, CHUNK)
    emb = _sc_gather(table, idx3)
    emb3 = emb.reshape(B, L, E)
    return _tc_project(emb3, W, b)


# speedup vs baseline: 5.0565x; 1.1699x over previous
import functools

import jax
import jax.numpy as jnp
from jax import lax
from jax.experimental import pallas as pl
from jax.experimental.pallas import tpu as pltpu
from jax.experimental.pallas import tpu_sc as plsc

B, C, L = 256, 64, 512
E = 16
NPAT = 1000000
HB = 8

NC, NS = 2, 16
NW = NC * NS
ROWS = B * L
RPW = ROWS // NW
CHUNK = 128
NCHUNK = RPW // CHUNK
GROUP = 8


def _shift3(y, d):
    if d < 0:
        return jnp.concatenate([y[:, :, :1]] * (-d) + [y[:, :, :L + d]], axis=2)
    if d > 0:
        return jnp.concatenate([y[:, :, d:]] + [y[:, :, L - 1:]] * d, axis=2)
    return y


def _hash_body(x_ref, out_ref):
    y = x_ref[...] * 31.0
    p = jnp.zeros((HB, 8, L), jnp.float32)
    for k in range(8):
        yk = y[:, 8 * k:8 * k + 8, :]
        for d in (-2, -1, 0, 1):
            p = p + _shift3(yk, d)
    q = p[:, 0:4, :] + p[:, 4:8, :]
    r = q[:, 0:2, :] + q[:, 2:4, :]
    h = r[:, 0:1, :] + r[:, 1:2, :]
    hi = h[:, 0, :].astype(jnp.int32)
    m = lax.rem(hi, jnp.int32(NPAT))
    out_ref[...] = jnp.where(m < 0, m + NPAT, m)


def _tc_hash(x):
    return pl.pallas_call(
        _hash_body,
        grid=(B // HB,),
        in_specs=[pl.BlockSpec((HB, C, L), lambda i: (i, 0, 0))],
        out_specs=pl.BlockSpec((HB, L), lambda i: (i, 0)),
        out_shape=jax.ShapeDtypeStruct((B, L), jnp.int32),
    )(x)


def _sc_gather(table, idx3):
    mesh = plsc.VectorSubcoreMesh(core_axis_name="c", subcore_axis_name="s")

    @functools.partial(
        pl.kernel,
        out_type=jax.ShapeDtypeStruct((ROWS, E), jnp.float32),
        mesh=mesh,
        scratch_types=[
            pltpu.VMEM((NCHUNK, CHUNK), jnp.int32),
            pltpu.VMEM((RPW, E), jnp.float32),
            pltpu.SemaphoreType.DMA,
        ],
        compiler_params=pltpu.CompilerParams(use_tc_tiling_on_sc=False),
    )
    def gather_k(table_hbm, idx_hbm, out_hbm, idx_v, rows_v, sem):
        wid = lax.axis_index("s") * NC + lax.axis_index("c")
        pltpu.sync_copy(idx_hbm.at[wid], idx_v)

        def group_body(g, _):
            base_c = g * GROUP
            copies = []
            for k in range(GROUP):
                c = base_c + k
                cp = pltpu.make_async_copy(
                    table_hbm.at[idx_v.at[c]],
                    rows_v.at[pl.ds(c * CHUNK, CHUNK), :],
                    sem,
                )
                cp.start()
                copies.append(cp)
            for cp in copies:
                cp.wait()
            return 0

        lax.fori_loop(0, NCHUNK // GROUP, group_body, 0, unroll=False)
        pltpu.sync_copy(rows_v, out_hbm.at[pl.ds(wid * RPW, RPW), :])

    return gather_k(table, idx3)


def _proj_body(emb_ref, w_ref, b_ref, out_ref):
    e = emb_ref[...]
    w = w_ref[...]
    for k in range(8):
        ek = e[:, 16 * k:16 * k + 16]
        rk = lax.dot_general(w, ek, (((1,), (1,)), ((), ())),
                             preferred_element_type=jnp.float32)
        out_ref[0, :, 64 * k:64 * k + 64] = rk + b_ref[...]


def _tc_project(emb2, W, b):
    return pl.pallas_call(
        _proj_body,
        grid=(B,),
        in_specs=[
            pl.BlockSpec((C, 128), lambda i: (i, 0)),
            pl.BlockSpec((C, E), lambda i: (0, 0)),
            pl.BlockSpec((C, 1), lambda i: (0, 0)),
        ],
        out_specs=pl.BlockSpec((1, C, L), lambda i: (i, 0, 0)),
        out_shape=jax.ShapeDtypeStruct((B, C, L), jnp.float32),
    )(emb2, W, b.reshape(C, 1))


def kernel(x, table, W, b):
    idx = _tc_hash(x)
    idx_perm = idx.reshape(B, 8, 64).transpose(0, 2, 1)
    idx3 = idx_perm.reshape(NW, NCHUNK, CHUNK)
    emb = _sc_gather(table, idx3)
    emb2 = emb.reshape(ROWS // 8, 128)
    return _tc_project(emb2, W, b)


# speedup vs baseline: 5.9680x; 1.1803x over previous
import functools

import jax
import jax.numpy as jnp
from jax import lax
from jax.experimental import pallas as pl
from jax.experimental.pallas import tpu as pltpu
from jax.experimental.pallas import tpu_sc as plsc

B, C, L = 256, 64, 512
E = 16
NPAT = 1000000
HB = 8

NC, NS = 2, 16
NW = NC * NS
ROWS = B * L
RPW = ROWS // NW
CHUNK = 128
NCHUNK = RPW // CHUNK
GROUP = 8


def _shift3(y, d):
    if d < 0:
        return jnp.concatenate([y[:, :, :1]] * (-d) + [y[:, :, :L + d]], axis=2)
    if d > 0:
        return jnp.concatenate([y[:, :, d:]] + [y[:, :, L - 1:]] * d, axis=2)
    return y


def _hash_body(x_ref, out_ref):
    y = x_ref[...] * 31.0
    p = jnp.zeros((HB, 8, L), jnp.float32)
    for k in range(8):
        yk = y[:, 8 * k:8 * k + 8, :]
        for d in (-2, -1, 0, 1):
            p = p + _shift3(yk, d)
    q = p[:, 0:4, :] + p[:, 4:8, :]
    r = q[:, 0:2, :] + q[:, 2:4, :]
    h = r[:, 0:1, :] + r[:, 1:2, :]
    hi = h[:, 0, :].astype(jnp.int32)
    m = lax.rem(hi, jnp.int32(NPAT))
    out_ref[...] = jnp.where(m < 0, m + NPAT, m)


def _tc_hash(x):
    return pl.pallas_call(
        _hash_body,
        grid=(B // HB,),
        in_specs=[pl.BlockSpec((HB, C, L), lambda i: (i, 0, 0))],
        out_specs=pl.BlockSpec((HB, L), lambda i: (i, 0)),
        out_shape=jax.ShapeDtypeStruct((B, L), jnp.int32),
    )(x)


NPATR = NPAT // 8
RBLK = 8192
RGRID = -(-NPAT // RBLK)


def _repack_body(tt_ref, out_ref, scr_ref):
    scr_ref[...] = jnp.transpose(tt_ref[...], (1, 0))
    for k in range(8):
        out_ref[:, E * k:E * k + E] = scr_ref[pl.Slice(k, RBLK // 8, 8), :]


def _tc_repack(tableT):
    return pl.pallas_call(
        _repack_body,
        grid=(RGRID,),
        in_specs=[pl.BlockSpec((E, RBLK), lambda i: (0, i))],
        out_specs=pl.BlockSpec((RBLK // 8, 128), lambda i: (i, 0)),
        out_shape=jax.ShapeDtypeStruct((NPATR, 128), jnp.float32),
        scratch_shapes=[pltpu.VMEM((RBLK, E), jnp.float32)],
    )(tableT)


def _sc_gather(table, idx3):
    mesh = plsc.VectorSubcoreMesh(core_axis_name="c", subcore_axis_name="s")

    @functools.partial(
        pl.kernel,
        out_type=jax.ShapeDtypeStruct((ROWS, E), jnp.float32),
        mesh=mesh,
        scratch_types=[
            pltpu.VMEM((NCHUNK, CHUNK), jnp.int32),
            pltpu.VMEM((RPW, E), jnp.float32),
            pltpu.SemaphoreType.DMA,
        ],
        compiler_params=pltpu.CompilerParams(use_tc_tiling_on_sc=False),
    )
    def gather_k(table_hbm, idx_hbm, out_hbm, idx_v, rows_v, sem):
        wid = lax.axis_index("s") * NC + lax.axis_index("c")
        pltpu.sync_copy(idx_hbm.at[wid], idx_v)

        def group_body(g, _):
            base_c = g * GROUP
            copies = []
            for k in range(GROUP):
                c = base_c + k
                cp = pltpu.make_async_copy(
                    table_hbm.at[idx_v.at[c]],
                    rows_v.at[pl.ds(c * CHUNK, CHUNK), :],
                    sem,
                )
                cp.start()
                copies.append(cp)
            for cp in copies:
                cp.wait()
            return 0

        lax.fori_loop(0, NCHUNK // GROUP, group_body, 0, unroll=False)
        pltpu.sync_copy(rows_v, out_hbm.at[pl.ds(wid * RPW, RPW), :])

    return gather_k(table, idx3)


def _proj_body(emb_ref, w_ref, b_ref, out_ref):
    e = emb_ref[...]
    w = w_ref[...]
    for k in range(8):
        ek = e[:, 16 * k:16 * k + 16]
        rk = lax.dot_general(w, ek, (((1,), (1,)), ((), ())),
                             preferred_element_type=jnp.float32)
        out_ref[0, :, 64 * k:64 * k + 64] = rk + b_ref[...]


def _tc_project(emb2, W, b):
    return pl.pallas_call(
        _proj_body,
        grid=(B,),
        in_specs=[
            pl.BlockSpec((C, 128), lambda i: (i, 0)),
            pl.BlockSpec((C, E), lambda i: (0, 0)),
            pl.BlockSpec((C, 1), lambda i: (0, 0)),
        ],
        out_specs=pl.BlockSpec((1, C, L), lambda i: (i, 0, 0)),
        out_shape=jax.ShapeDtypeStruct((B, C, L), jnp.float32),
    )(emb2, W, b.reshape(C, 1))


def kernel(x, table, W, b):
    idx = _tc_hash(x)
    idx_perm = idx.reshape(B, 8, 64).transpose(0, 2, 1)
    idx3 = idx_perm.reshape(NW, NCHUNK, CHUNK)
    tbl_lin = _tc_repack(table.T).reshape(NPAT, E)
    emb = _sc_gather(tbl_lin, idx3)
    emb2 = emb.reshape(ROWS // 8, 128)
    return _tc_project(emb2, W, b)


# speedup vs baseline: 9.4515x; 1.5837x over previous
import functools

import jax
import jax.numpy as jnp
from jax import lax
from jax.experimental import pallas as pl
from jax.experimental.pallas import tpu as pltpu
from jax.experimental.pallas import tpu_sc as plsc

B, C, L = 256, 64, 512
E = 16
NPAT = 1000000
HB = 8

NC, NS = 2, 16
NW = NC * NS
ROWS = B * L
RPW = ROWS // NW
CHUNK = 128
NCHUNK = RPW // CHUNK
GROUP = 8


def _shift3(y, d):
    if d < 0:
        return jnp.concatenate([y[:, :, :1]] * (-d) + [y[:, :, :L + d]], axis=2)
    if d > 0:
        return jnp.concatenate([y[:, :, d:]] + [y[:, :, L - 1:]] * d, axis=2)
    return y


def _hash_body(x_ref, out_ref):
    y = x_ref[...] * 31.0
    p = jnp.zeros((HB, 8, L), jnp.float32)
    for k in range(8):
        yk = y[:, 8 * k:8 * k + 8, :]
        for d in (-2, -1, 0, 1):
            p = p + _shift3(yk, d)
    q = p[:, 0:4, :] + p[:, 4:8, :]
    r = q[:, 0:2, :] + q[:, 2:4, :]
    h = r[:, 0:1, :] + r[:, 1:2, :]
    hi = h[:, 0, :].astype(jnp.int32)
    m = lax.rem(hi, jnp.int32(NPAT))
    m = jnp.where(m < 0, m + NPAT, m)
    out_ref[...] = ((m >> 10) << 10) + ((m & 127) << 3) + ((m >> 7) & 7)


def _tc_hash(x):
    return pl.pallas_call(
        _hash_body,
        grid=(B // HB,),
        in_specs=[pl.BlockSpec((HB, C, L), lambda i: (i, 0, 0))],
        out_specs=pl.BlockSpec((HB, L), lambda i: (i, 0)),
        out_shape=jax.ShapeDtypeStruct((B, L), jnp.int32),
    )(x)


RBLK = 8192
RGRID = -(-NPAT // RBLK)
NSLOT = RGRID * RBLK // 8


def _repack_body(tt_ref, out_ref):
    blk = tt_ref[...]
    for g8 in range(8):
        base = 1024 * g8
        stacked = jnp.concatenate(
            [blk[:, base + 128 * a:base + 128 * (a + 1)] for a in range(8)],
            axis=0)
        out_ref[128 * g8:128 * (g8 + 1), :] = jnp.transpose(stacked, (1, 0))


def _tc_repack(tableT):
    return pl.pallas_call(
        _repack_body,
        grid=(RGRID,),
        in_specs=[pl.BlockSpec((E, RBLK), lambda i: (0, i))],
        out_specs=pl.BlockSpec((RBLK // 8, 128), lambda i: (i, 0)),
        out_shape=jax.ShapeDtypeStruct((NSLOT, 128), jnp.float32),
    )(tableT)


def _sc_gather(table, idx3):
    mesh = plsc.VectorSubcoreMesh(core_axis_name="c", subcore_axis_name="s")

    @functools.partial(
        pl.kernel,
        out_type=jax.ShapeDtypeStruct((ROWS, E), jnp.float32),
        mesh=mesh,
        scratch_types=[
            pltpu.VMEM((NCHUNK, CHUNK), jnp.int32),
            pltpu.VMEM((RPW, E), jnp.float32),
            pltpu.SemaphoreType.DMA,
        ],
        compiler_params=pltpu.CompilerParams(use_tc_tiling_on_sc=False),
    )
    def gather_k(table_hbm, idx_hbm, out_hbm, idx_v, rows_v, sem):
        wid = lax.axis_index("s") * NC + lax.axis_index("c")
        pltpu.sync_copy(idx_hbm.at[wid], idx_v)

        def group_body(g, _):
            base_c = g * GROUP
            copies = []
            for k in range(GROUP):
                c = base_c + k
                cp = pltpu.make_async_copy(
                    table_hbm.at[idx_v.at[c]],
                    rows_v.at[pl.ds(c * CHUNK, CHUNK), :],
                    sem,
                )
                cp.start()
                copies.append(cp)
            for cp in copies:
                cp.wait()
            return 0

        lax.fori_loop(0, NCHUNK // GROUP, group_body, 0, unroll=False)
        pltpu.sync_copy(rows_v, out_hbm.at[pl.ds(wid * RPW, RPW), :])

    return gather_k(table, idx3)


def _proj_body(emb_ref, w_ref, b_ref, out_ref):
    e = emb_ref[...]
    w = w_ref[...]
    for k in range(8):
        ek = e[:, 16 * k:16 * k + 16]
        rk = lax.dot_general(w, ek, (((1,), (1,)), ((), ())),
                             preferred_element_type=jnp.float32)
        out_ref[0, :, 64 * k:64 * k + 64] = rk + b_ref[...]


def _tc_project(emb2, W, b):
    return pl.pallas_call(
        _proj_body,
        grid=(B,),
        in_specs=[
            pl.BlockSpec((C, 128), lambda i: (i, 0)),
            pl.BlockSpec((C, E), lambda i: (0, 0)),
            pl.BlockSpec((C, 1), lambda i: (0, 0)),
        ],
        out_specs=pl.BlockSpec((1, C, L), lambda i: (i, 0, 0)),
        out_shape=jax.ShapeDtypeStruct((B, C, L), jnp.float32),
    )(emb2, W, b.reshape(C, 1))


def kernel(x, table, W, b):
    idx = _tc_hash(x)
    idx_perm = idx.reshape(B, 8, 64).transpose(0, 2, 1)
    idx3 = idx_perm.reshape(NW, NCHUNK, CHUNK)
    tbl_lin = _tc_repack(table.T).reshape(NSLOT * 8, E)
    emb = _sc_gather(tbl_lin, idx3)
    emb2 = emb.reshape(ROWS // 8, 128)
    return _tc_project(emb2, W, b)


# speedup vs baseline: 9.4838x; 1.0034x over previous
import functools

import jax
import jax.numpy as jnp
from jax import lax
from jax.experimental import pallas as pl
from jax.experimental.pallas import tpu as pltpu
from jax.experimental.pallas import tpu_sc as plsc

B, C, L = 256, 64, 512
E = 16
NPAT = 1000000
HB = 8

NC, NS = 2, 16
NW = NC * NS
ROWS = B * L
RPW = ROWS // NW
CHUNK = 128
NCHUNK = RPW // CHUNK
GROUP = 8


def _shift3(y, d):
    if d < 0:
        return jnp.concatenate([y[:, :, :1]] * (-d) + [y[:, :, :L + d]], axis=2)
    if d > 0:
        return jnp.concatenate([y[:, :, d:]] + [y[:, :, L - 1:]] * d, axis=2)
    return y


def _hash_body(x_ref, out_ref):
    y = x_ref[...] * 31.0
    p = jnp.zeros((HB, 8, L), jnp.float32)
    for k in range(8):
        yk = y[:, 8 * k:8 * k + 8, :]
        for d in (-2, -1, 0, 1):
            p = p + _shift3(yk, d)
    q = p[:, 0:4, :] + p[:, 4:8, :]
    r = q[:, 0:2, :] + q[:, 2:4, :]
    h = r[:, 0:1, :] + r[:, 1:2, :]
    hi = h[:, 0, :].astype(jnp.int32)
    m = lax.rem(hi, jnp.int32(NPAT))
    m = jnp.where(m < 0, m + NPAT, m)
    out_ref[...] = ((m >> 10) << 10) + ((m & 127) << 3) + ((m >> 7) & 7)


def _tc_hash(x):
    return pl.pallas_call(
        _hash_body,
        grid=(B // HB,),
        in_specs=[pl.BlockSpec((HB, C, L), lambda i: (i, 0, 0))],
        out_specs=pl.BlockSpec((HB, L), lambda i: (i, 0)),
        out_shape=jax.ShapeDtypeStruct((B, L), jnp.int32),
    )(x)


RBLK = 8192
RGRID = -(-NPAT // RBLK)
NSLOT = RGRID * RBLK // 8


def _repack_body(tt_ref, out_ref):
    blk = tt_ref[...]
    for g8 in range(8):
        base = 1024 * g8
        stacked = jnp.concatenate(
            [blk[:, base + 128 * a:base + 128 * (a + 1)] for a in range(8)],
            axis=0)
        out_ref[128 * g8:128 * (g8 + 1), :] = jnp.transpose(stacked, (1, 0))


def _tc_repack(tableT):
    return pl.pallas_call(
        _repack_body,
        grid=(RGRID,),
        in_specs=[pl.BlockSpec((E, RBLK), lambda i: (0, i))],
        out_specs=pl.BlockSpec((RBLK // 8, 128), lambda i: (i, 0)),
        out_shape=jax.ShapeDtypeStruct((NSLOT, 128), jnp.float32),
    )(tableT)


def _sc_gather(table, idx3):
    mesh = plsc.VectorSubcoreMesh(core_axis_name="c", subcore_axis_name="s")

    @functools.partial(
        pl.kernel,
        out_type=jax.ShapeDtypeStruct((ROWS, E), jnp.float32),
        mesh=mesh,
        scratch_types=[
            pltpu.VMEM((NCHUNK, CHUNK), jnp.int32),
            pltpu.VMEM((RPW, E), jnp.float32),
            pltpu.SemaphoreType.DMA,
        ],
        compiler_params=pltpu.CompilerParams(use_tc_tiling_on_sc=False),
    )
    def gather_k(table_hbm, idx_hbm, out_hbm, idx_v, rows_v, sem):
        wid = lax.axis_index("s") * NC + lax.axis_index("c")
        pltpu.sync_copy(idx_hbm.at[wid], idx_v)

        def group_body(g, _):
            base_c = g * GROUP
            copies = []
            for k in range(GROUP):
                c = base_c + k
                cp = pltpu.make_async_copy(
                    table_hbm.at[idx_v.at[c]],
                    rows_v.at[pl.ds(c * CHUNK, CHUNK), :],
                    sem,
                )
                cp.start()
                copies.append(cp)
            for cp in copies:
                cp.wait()
            return 0

        lax.fori_loop(0, NCHUNK // GROUP, group_body, 0, unroll=False)
        pltpu.sync_copy(rows_v, out_hbm.at[pl.ds(wid * RPW, RPW), :])

    return gather_k(table, idx3)


def _proj_body(emb_ref, w_ref, b_ref, out_ref):
    e = emb_ref[...]
    t = jnp.transpose(e, (1, 0))
    x = jnp.concatenate([t[16 * k:16 * k + 16, :] for k in range(8)],
                        axis=1)
    r = lax.dot_general(w_ref[...], x, (((1,), (0,)), ((), ())),
                        preferred_element_type=jnp.float32)
    out_ref[0] = r + b_ref[...]


def _tc_project(emb2, W, b):
    return pl.pallas_call(
        _proj_body,
        grid=(B,),
        in_specs=[
            pl.BlockSpec((C, 128), lambda i: (i, 0)),
            pl.BlockSpec((C, E), lambda i: (0, 0)),
            pl.BlockSpec((C, 1), lambda i: (0, 0)),
        ],
        out_specs=pl.BlockSpec((1, C, L), lambda i: (i, 0, 0)),
        out_shape=jax.ShapeDtypeStruct((B, C, L), jnp.float32),
    )(emb2, W, b.reshape(C, 1))


def kernel(x, table, W, b):
    idx = _tc_hash(x)
    idx_perm = idx.reshape(B, 8, 64).transpose(0, 2, 1)
    idx3 = idx_perm.reshape(NW, NCHUNK, CHUNK)
    tbl_lin = _tc_repack(table.T).reshape(NSLOT * 8, E)
    emb = _sc_gather(tbl_lin, idx3)
    emb2 = emb.reshape(ROWS // 8, 128)
    return _tc_project(emb2, W, b)


# speedup vs baseline: 16.2329x; 1.7116x over previous
import functools

import jax
import jax.numpy as jnp
from jax import lax
from jax.experimental import pallas as pl
from jax.experimental.pallas import tpu as pltpu
from jax.experimental.pallas import tpu_sc as plsc

B, C, L = 256, 64, 512
E = 16
NPAT = 1000000
HB = 16

NC, NS = 2, 16
NW = NC * NS
ROWS = B * L
RPW = ROWS // NW
CHUNK = 128
NCHUNK = RPW // CHUNK
GROUP = 8


def _shift3(y, d):
    if d < 0:
        return jnp.concatenate([y[:, :, :1]] * (-d) + [y[:, :, :L + d]], axis=2)
    if d > 0:
        return jnp.concatenate([y[:, :, d:]] + [y[:, :, L - 1:]] * d, axis=2)
    return y


def _hash_body(x_ref, out_ref):
    y = x_ref[...] * 31.0
    p = jnp.zeros((HB, 8, L), jnp.float32)
    for k in range(8):
        yk = y[:, 8 * k:8 * k + 8, :]
        for d in (-2, -1, 0, 1):
            p = p + _shift3(yk, d)
    q = p[:, 0:4, :] + p[:, 4:8, :]
    r = q[:, 0:2, :] + q[:, 2:4, :]
    h = r[:, 0:1, :] + r[:, 1:2, :]
    hi = h[:, 0, :].astype(jnp.int32)
    m = lax.rem(hi, jnp.int32(NPAT))
    m = jnp.where(m < 0, m + NPAT, m)
    out_ref[...] = ((m >> 10) << 10) + ((m & 127) << 3) + ((m >> 7) & 7)


def _tc_hash(x):
    return pl.pallas_call(
        _hash_body,
        grid=(B // HB,),
        in_specs=[pl.BlockSpec((HB, C, L), lambda i: (i, 0, 0))],
        out_specs=pl.BlockSpec((HB, L), lambda i: (i, 0)),
        out_shape=jax.ShapeDtypeStruct((B, L), jnp.int32),
    )(x)


RBLK = 16384
RGRID = -(-NPAT // RBLK)
NSLOT = RGRID * RBLK // 8


def _repack_body(tt_ref, out_ref):
    blk = tt_ref[...]
    for g8 in range(RBLK // 1024):
        base = 1024 * g8
        stacked = jnp.concatenate(
            [blk[:, base + 128 * a:base + 128 * (a + 1)] for a in range(8)],
            axis=0)
        out_ref[128 * g8:128 * (g8 + 1), :] = jnp.transpose(stacked, (1, 0))


def _tc_repack(tableT):
    return pl.pallas_call(
        _repack_body,
        grid=(RGRID,),
        in_specs=[pl.BlockSpec((E, RBLK), lambda i: (0, i))],
        out_specs=pl.BlockSpec((RBLK // 8, 128), lambda i: (i, 0)),
        out_shape=jax.ShapeDtypeStruct((NSLOT, 128), jnp.float32),
    )(tableT)


def _sc_gather(table, idx3):
    mesh = plsc.VectorSubcoreMesh(core_axis_name="c", subcore_axis_name="s")

    @functools.partial(
        pl.kernel,
        out_type=jax.ShapeDtypeStruct((ROWS, E), jnp.float32),
        mesh=mesh,
        scratch_types=[
            pltpu.VMEM((NCHUNK, CHUNK), jnp.int32),
            pltpu.VMEM((RPW, E), jnp.float32),
            pltpu.SemaphoreType.DMA,
        ],
        compiler_params=pltpu.CompilerParams(use_tc_tiling_on_sc=False),
    )
    def gather_k(table_hbm, idx_hbm, out_hbm, idx_v, rows_v, sem):
        wid = lax.axis_index("s") * NC + lax.axis_index("c")
        pltpu.sync_copy(idx_hbm.at[wid], idx_v)

        def group_body(g, _):
            base_c = g * GROUP
            copies = []
            for k in range(GROUP):
                c = base_c + k
                cp = pltpu.make_async_copy(
                    table_hbm.at[idx_v.at[c]],
                    rows_v.at[pl.ds(c * CHUNK, CHUNK), :],
                    sem,
                )
                cp.start()
                copies.append(cp)
            for cp in copies:
                cp.wait()
            return 0

        lax.fori_loop(0, NCHUNK // GROUP, group_body, 0, unroll=False)
        pltpu.sync_copy(rows_v, out_hbm.at[pl.ds(wid * RPW, RPW), :])

    return gather_k(table, idx3)


PB = 4


def _proj_body(emb_ref, w_ref, b_ref, out_ref):
    t = jnp.transpose(emb_ref[...], (1, 0))
    for m in range(PB):
        x = jnp.concatenate(
            [t[16 * k:16 * k + 16, 64 * m:64 * m + 64] for k in range(8)],
            axis=1)
        r = lax.dot_general(w_ref[...], x, (((1,), (0,)), ((), ())),
                            preferred_element_type=jnp.float32)
        out_ref[m] = r + b_ref[...]


def _tc_project(emb2, W, b):
    return pl.pallas_call(
        _proj_body,
        grid=(B // PB,),
        in_specs=[
            pl.BlockSpec((PB * C, 128), lambda i: (i, 0)),
            pl.BlockSpec((C, E), lambda i: (0, 0)),
            pl.BlockSpec((C, 1), lambda i: (0, 0)),
        ],
        out_specs=pl.BlockSpec((PB, C, L), lambda i: (i, 0, 0)),
        out_shape=jax.ShapeDtypeStruct((B, C, L), jnp.float32),
    )(emb2, W, b.reshape(C, 1))


def kernel(x, table, W, b):
    idx = _tc_hash(x)
    idx_perm = idx.reshape(B, 8, 64).transpose(0, 2, 1)
    idx3 = idx_perm.reshape(NW, NCHUNK, CHUNK)
    tbl_lin = _tc_repack(table.T).reshape(NSLOT * 8, E)
    emb = _sc_gather(tbl_lin, idx3)
    emb2 = emb.reshape(ROWS // 8, 128)
    return _tc_project(emb2, W, b)


# speedup vs baseline: 19.5412x; 1.2038x over previous
import functools

import jax
import jax.numpy as jnp
from jax import lax
from jax.experimental import pallas as pl
from jax.experimental.pallas import tpu as pltpu
from jax.experimental.pallas import tpu_sc as plsc

B, C, L = 256, 64, 512
E = 16
NPAT = 1000000
HB = 32

NC, NS = 2, 16
NW = NC * NS
ROWS = B * L
RPW = ROWS // NW
CHUNK = 128
NCHUNK = RPW // CHUNK
GROUP = 8


def _shift3(y, d):
    if d < 0:
        return jnp.concatenate([y[:, :, :1]] * (-d) + [y[:, :, :L + d]], axis=2)
    if d > 0:
        return jnp.concatenate([y[:, :, d:]] + [y[:, :, L - 1:]] * d, axis=2)
    return y


def _hash_body(x_ref, out_ref):
    y = x_ref[...] * 31.0
    p = jnp.zeros((HB, 8, L), jnp.float32)
    for k in range(8):
        yk = y[:, 8 * k:8 * k + 8, :]
        for d in (-2, -1, 0, 1):
            p = p + _shift3(yk, d)
    q = p[:, 0:4, :] + p[:, 4:8, :]
    r = q[:, 0:2, :] + q[:, 2:4, :]
    h = r[:, 0:1, :] + r[:, 1:2, :]
    hi = h[:, 0, :].astype(jnp.int32)
    m = lax.rem(hi, jnp.int32(NPAT))
    m = jnp.where(m < 0, m + NPAT, m)
    out_ref[...] = ((m >> 10) << 10) + ((m & 127) << 3) + ((m >> 7) & 7)


def _tc_hash(x):
    return pl.pallas_call(
        _hash_body,
        grid=(B // HB,),
        in_specs=[pl.BlockSpec((HB, C, L), lambda i: (i, 0, 0))],
        out_specs=pl.BlockSpec((HB, L), lambda i: (i, 0)),
        out_shape=jax.ShapeDtypeStruct((B, L), jnp.int32),
    )(x)


RBLK = 32768
RGRID = -(-NPAT // RBLK)
NSLOT = RGRID * RBLK // 8


def _repack_body(tt_ref, out_ref):
    blk = tt_ref[...]
    for g8 in range(RBLK // 1024):
        base = 1024 * g8
        stacked = jnp.concatenate(
            [blk[:, base + 128 * a:base + 128 * (a + 1)] for a in range(8)],
            axis=0)
        out_ref[128 * g8:128 * (g8 + 1), :] = jnp.transpose(stacked, (1, 0))


def _tc_repack(tableT):
    return pl.pallas_call(
        _repack_body,
        grid=(RGRID,),
        in_specs=[pl.BlockSpec((E, RBLK), lambda i: (0, i))],
        out_specs=pl.BlockSpec((RBLK // 8, 128), lambda i: (i, 0)),
        out_shape=jax.ShapeDtypeStruct((NSLOT, 128), jnp.float32),
    )(tableT)


def _sc_gather(table, idx3):
    mesh = plsc.VectorSubcoreMesh(core_axis_name="c", subcore_axis_name="s")

    @functools.partial(
        pl.kernel,
        out_type=jax.ShapeDtypeStruct((ROWS, E), jnp.float32),
        mesh=mesh,
        scratch_types=[
            pltpu.VMEM((NCHUNK, CHUNK), jnp.int32),
            pltpu.VMEM((RPW, E), jnp.float32),
            pltpu.SemaphoreType.DMA,
        ],
        compiler_params=pltpu.CompilerParams(use_tc_tiling_on_sc=False),
    )
    def gather_k(table_hbm, idx_hbm, out_hbm, idx_v, rows_v, sem):
        wid = lax.axis_index("s") * NC + lax.axis_index("c")
        pltpu.sync_copy(idx_hbm.at[wid], idx_v)

        def group_body(g, _):
            base_c = g * GROUP
            copies = []
            for k in range(GROUP):
                c = base_c + k
                cp = pltpu.make_async_copy(
                    table_hbm.at[idx_v.at[c]],
                    rows_v.at[pl.ds(c * CHUNK, CHUNK), :],
                    sem,
                )
                cp.start()
                copies.append(cp)
            for cp in copies:
                cp.wait()
            return 0

        lax.fori_loop(0, NCHUNK // GROUP, group_body, 0, unroll=False)
        pltpu.sync_copy(rows_v, out_hbm.at[pl.ds(wid * RPW, RPW), :])

    return gather_k(table, idx3)


PB = 8


def _proj_body(emb_ref, w_ref, b_ref, out_ref):
    t = jnp.transpose(emb_ref[...], (1, 0))
    for m in range(PB):
        x = jnp.concatenate(
            [t[16 * k:16 * k + 16, 64 * m:64 * m + 64] for k in range(8)],
            axis=1)
        r = lax.dot_general(w_ref[...], x, (((1,), (0,)), ((), ())),
                            preferred_element_type=jnp.float32)
        out_ref[m] = r + b_ref[...]


def _tc_project(emb2, W, b):
    return pl.pallas_call(
        _proj_body,
        grid=(B // PB,),
        in_specs=[
            pl.BlockSpec((PB * C, 128), lambda i: (i, 0)),
            pl.BlockSpec((C, E), lambda i: (0, 0)),
            pl.BlockSpec((C, 1), lambda i: (0, 0)),
        ],
        out_specs=pl.BlockSpec((PB, C, L), lambda i: (i, 0, 0)),
        out_shape=jax.ShapeDtypeStruct((B, C, L), jnp.float32),
    )(emb2, W, b.reshape(C, 1))


def kernel(x, table, W, b):
    idx = _tc_hash(x)
    idx_perm = idx.reshape(B, 8, 64).transpose(0, 2, 1)
    idx3 = idx_perm.reshape(NW, NCHUNK, CHUNK)
    tbl_lin = _tc_repack(table.T).reshape(NSLOT * 8, E)
    emb = _sc_gather(tbl_lin, idx3)
    emb2 = emb.reshape(ROWS // 8, 128)
    return _tc_project(emb2, W, b)
